# Initial kernel scaffold; baseline (speedup 1.0000x reference)
import functools

import jax
import jax.numpy as jnp
from jax import lax
from jax.experimental import pallas as pl
from jax.experimental.pallas import tpu as pltpu
from jax.experimental.pallas import tpu_sc as plsc

_N = 4096
_N_DEPTH = 3
_N_ORI = 100
_N_GEN = 10
_N_BINS = 50
_EPD = _N_DEPTH * _N_ORI * _N_GEN
_NW = 32
_DOCS_PER_W = _N // _NW
_DOCS_PER_CHUNK = 16
_CHUNK_WORDS = _DOCS_PER_CHUNK * _EPD
_NCHUNK = _DOCS_PER_W // _DOCS_PER_CHUNK
_TAB = 160


def _sc_body(images, wtab_h, params_h, out0_h, out1_h,
             wtab, params, buf0, buf1, sums, row0, row1, sem0, sem1):
    c = lax.axis_index("c")
    s = lax.axis_index("s")
    wid = s * 2 + c
    base = wid * (_DOCS_PER_W * _EPD)

    pltpu.sync_copy(wtab_h, wtab)
    pltpu.sync_copy(params_h, params)

    bufs = (buf0, buf1)
    sems = (sem0, sem1)
    handles = [None, None]
    handles[0] = pltpu.async_copy(
        images.at[pl.ds(base, _CHUNK_WORDS)], bufs[0], sems[0])

    lanes = lax.iota(jnp.int32, 16)
    doc_base = lanes * _EPD

    for ck in range(_NCHUNK):
        b = ck & 1
        handles[b].wait()
        if ck + 1 < _NCHUNK:
            handles[1 - b] = pltpu.async_copy(
                images.at[pl.ds(base + (ck + 1) * _CHUNK_WORDS, _CHUNK_WORDS)],
                bufs[1 - b], sems[1 - b])

        buf = bufs[b]
        acc = jnp.zeros((16,), jnp.float32)
        idxv = doc_base

        for d in range(_N_DEPTH):
            off = d * _N_BINS

            def step(j, carry, _buf=buf, _off=off):
                a, iv = carry
                x = plsc.load_gather(_buf, [iv])
                t = x * jnp.float32(_N_BINS)
                ti = jnp.minimum(t.astype(jnp.int32), _N_BINS - 1)
                if _off:
                    ti = ti + _off
                w = plsc.load_gather(wtab, [ti])
                return a + w, iv + 1

            acc, idxv = lax.fori_loop(0, _N_ORI * _N_GEN, step, (acc, idxv))

        sums[pl.ds(ck * 16, 16)] = acc

    a0 = params[pl.ds(0, 16)]
    a1 = params[pl.ds(16, 16)]
    c0 = params[pl.ds(32, 16)]
    c1 = params[pl.ds(48, 16)]
    for g in range(_NCHUNK):
        sv = sums[pl.ds(g * 16, 16)]
        row0[pl.ds(g * 16, 16)] = sv * a0 + c0
        row1[pl.ds(g * 16, 16)] = sv * a1 + c1

    pltpu.sync_copy(row0, out0_h.at[pl.ds(wid * _DOCS_PER_W, _DOCS_PER_W)])
    pltpu.sync_copy(row1, out1_h.at[pl.ds(wid * _DOCS_PER_W, _DOCS_PER_W)])


_mesh = plsc.VectorSubcoreMesh(core_axis_name="c", subcore_axis_name="s")

_sc_call = functools.partial(
    pl.kernel,
    mesh=_mesh,
    out_type=[
        jax.ShapeDtypeStruct((_N,), jnp.float32),
        jax.ShapeDtypeStruct((_N,), jnp.float32),
    ],
    scratch_types=[
        pltpu.VMEM((_TAB,), jnp.float32),
        pltpu.VMEM((64,), jnp.float32),
        pltpu.VMEM((_CHUNK_WORDS,), jnp.float32),
        pltpu.VMEM((_CHUNK_WORDS,), jnp.float32),
        pltpu.VMEM((_DOCS_PER_W,), jnp.float32),
        pltpu.VMEM((_DOCS_PER_W,), jnp.float32),
        pltpu.VMEM((_DOCS_PER_W,), jnp.float32),
        pltpu.SemaphoreType.DMA,
        pltpu.SemaphoreType.DMA,
    ],
)(_sc_body)


def kernel(images, W_mlp, b_mlp, W_final, b_final):
    flat = images.reshape(-1)
    wtab = jnp.concatenate(
        [W_mlp[:, 0], jnp.zeros((_TAB - _N_DEPTH * _N_BINS,), jnp.float32)])
    wsum = W_final[0] + W_final[1] + W_final[2]
    a = wsum / jnp.float32(_N_GEN)
    cc = b_mlp[0] * wsum + b_final
    params = jnp.concatenate([
        jnp.full((16,), a[0], jnp.float32),
        jnp.full((16,), a[1], jnp.float32),
        jnp.full((16,), cc[0], jnp.float32),
        jnp.full((16,), cc[1], jnp.float32),
    ])
    out0, out1 = _sc_call(flat, wtab, params)
    return jnp.stack([out0, out1], axis=-1)


# speedup vs baseline: 28.3546x; 28.3546x over previous
import functools

import jax
import jax.numpy as jnp
from jax import lax
from jax.experimental import pallas as pl
from jax.experimental.pallas import tpu as pltpu
from jax.experimental.pallas import tpu_sc as plsc

_N = 4096
_N_DEPTH = 3
_N_ORI = 100
_N_GEN = 10
_N_BINS = 50
_EPD = _N_DEPTH * _N_ORI * _N_GEN
_NW = 32
_DOCS_PER_W = _N // _NW
_DOCS_PER_CHUNK = 16
_CHUNK_WORDS = _DOCS_PER_CHUNK * _EPD
_NCHUNK = _DOCS_PER_W // _DOCS_PER_CHUNK
_TAB = 160


def _sc_body(images, wtab_h, params_h, out0_h, out1_h,
             wtab, params, buf0, buf1, sums, row0, row1, sem0, sem1):
    c = lax.axis_index("c")
    s = lax.axis_index("s")
    wid = s * 2 + c
    base = wid * (_DOCS_PER_W * _EPD)

    pltpu.sync_copy(wtab_h, wtab)
    pltpu.sync_copy(params_h, params)

    bufs = (buf0, buf1)
    sems = (sem0, sem1)
    handles = [None, None]
    handles[0] = pltpu.async_copy(
        images.at[pl.ds(base, _CHUNK_WORDS)], bufs[0], sems[0])

    lanes = lax.iota(jnp.int32, 16)
    doc_base = lanes * _EPD

    for ck in range(_NCHUNK):
        b = ck & 1
        handles[b].wait()
        if ck + 1 < _NCHUNK:
            handles[1 - b] = pltpu.async_copy(
                images.at[pl.ds(base + (ck + 1) * _CHUNK_WORDS, _CHUNK_WORDS)],
                bufs[1 - b], sems[1 - b])

        buf = bufs[b]
        acc = jnp.zeros((16,), jnp.float32)
        idxv = doc_base

        for d in range(_N_DEPTH):
            off = d * _N_BINS

            def step(j, carry, _buf=buf, _off=off):
                a, iv = carry
                x = plsc.load_gather(_buf, [iv])
                t = x * jnp.float32(_N_BINS)
                ti = jnp.minimum(t.astype(jnp.int32), _N_BINS - 1)
                if _off:
                    ti = ti + _off
                w = plsc.load_gather(wtab, [ti])
                return a + w, iv + 1

            acc, idxv = lax.fori_loop(0, _N_ORI * _N_GEN, step, (acc, idxv))

        sums[pl.ds(ck * 16, 16)] = acc

    a0 = params[pl.ds(0, 16)]
    a1 = params[pl.ds(16, 16)]
    c0 = params[pl.ds(32, 16)]
    c1 = params[pl.ds(48, 16)]
    for g in range(_NCHUNK):
        sv = sums[pl.ds(g * 16, 16)]
        row0[pl.ds(g * 16, 16)] = sv * a0 + c0
        row1[pl.ds(g * 16, 16)] = sv * a1 + c1

    pltpu.sync_copy(row0, out0_h.at[pl.ds(wid * _DOCS_PER_W, _DOCS_PER_W)])
    pltpu.sync_copy(row1, out1_h.at[pl.ds(wid * _DOCS_PER_W, _DOCS_PER_W)])


_mesh = plsc.VectorSubcoreMesh(core_axis_name="c", subcore_axis_name="s")

_sc_call = functools.partial(
    pl.kernel,
    mesh=_mesh,
    compiler_params=pltpu.CompilerParams(needs_layout_passes=False),
    out_type=[
        jax.ShapeDtypeStruct((_N,), jnp.float32),
        jax.ShapeDtypeStruct((_N,), jnp.float32),
    ],
    scratch_types=[
        pltpu.VMEM((_TAB,), jnp.float32),
        pltpu.VMEM((64,), jnp.float32),
        pltpu.VMEM((_CHUNK_WORDS,), jnp.float32),
        pltpu.VMEM((_CHUNK_WORDS,), jnp.float32),
        pltpu.VMEM((_DOCS_PER_W,), jnp.float32),
        pltpu.VMEM((_DOCS_PER_W,), jnp.float32),
        pltpu.VMEM((_DOCS_PER_W,), jnp.float32),
        pltpu.SemaphoreType.DMA,
        pltpu.SemaphoreType.DMA,
    ],
)(_sc_body)


def kernel(images, W_mlp, b_mlp, W_final, b_final):
    flat = images.reshape(-1)
    wtab = jnp.concatenate(
        [W_mlp[:, 0], jnp.zeros((_TAB - _N_DEPTH * _N_BINS,), jnp.float32)])
    wsum = W_final[0] + W_final[1] + W_final[2]
    a = wsum / jnp.float32(_N_GEN)
    cc = b_mlp[0] * wsum + b_final
    params = jnp.concatenate([
        jnp.full((16,), a[0], jnp.float32),
        jnp.full((16,), a[1], jnp.float32),
        jnp.full((16,), cc[0], jnp.float32),
        jnp.full((16,), cc[1], jnp.float32),
    ])
    out0, out1 = _sc_call(flat, wtab, params)
    return jnp.stack([out0, out1], axis=-1)


# speedup vs baseline: 30.0863x; 1.0611x over previous
import functools

import jax
import jax.numpy as jnp
from jax import lax
from jax.experimental import pallas as pl
from jax.experimental.pallas import tpu as pltpu
from jax.experimental.pallas import tpu_sc as plsc

_N = 4096
_N_DEPTH = 3
_N_ORI = 100
_N_GEN = 10
_N_BINS = 50
_EPD = _N_DEPTH * _N_ORI * _N_GEN
_NW = 32
_DOCS_PER_W = _N // _NW
_DOCS_PER_CHUNK = 16
_CHUNK_WORDS = _DOCS_PER_CHUNK * _EPD
_NCHUNK = _DOCS_PER_W // _DOCS_PER_CHUNK
_TAB = 160


def _sc_body(images, wtab_h, params_h, out0_h, out1_h,
             wtab, params, buf0, buf1, sums, row0, row1, sem0, sem1):
    c = lax.axis_index("c")
    s = lax.axis_index("s")
    wid = s * 2 + c
    base = wid * (_DOCS_PER_W * _EPD)

    pltpu.sync_copy(wtab_h, wtab)
    pltpu.sync_copy(params_h, params)

    bufs = (buf0, buf1)
    sems = (sem0, sem1)
    handles = [None, None]
    handles[0] = pltpu.async_copy(
        images.at[pl.ds(base, _CHUNK_WORDS)], bufs[0], sems[0])

    lanes = lax.iota(jnp.int32, 16)
    doc_base = lanes * _EPD

    for ck in range(_NCHUNK):
        b = ck & 1
        handles[b].wait()
        if ck + 1 < _NCHUNK:
            handles[1 - b] = pltpu.async_copy(
                images.at[pl.ds(base + (ck + 1) * _CHUNK_WORDS, _CHUNK_WORDS)],
                bufs[1 - b], sems[1 - b])

        buf = bufs[b]
        acc = jnp.zeros((16,), jnp.float32)

        for d in range(_N_DEPTH):
            off = d * _N_BINS
            epd = _N_ORI * _N_GEN

            @plsc.parallel_loop(d * epd, (d + 1) * epd, unroll=8, carry=acc)
            def body(j, a, _buf=buf, _off=off):
                x = plsc.load_gather(_buf, [doc_base + j])
                t = jnp.minimum(x * jnp.float32(_N_BINS),
                                jnp.float32(_N_BINS - 1))
                ti = t.astype(jnp.int32)
                if _off:
                    ti = ti + _off
                w = plsc.load_gather(wtab, [ti])
                return a + w

            acc = body

        sums[pl.ds(ck * 16, 16)] = acc

    a0 = params[pl.ds(0, 16)]
    a1 = params[pl.ds(16, 16)]
    c0 = params[pl.ds(32, 16)]
    c1 = params[pl.ds(48, 16)]
    for g in range(_NCHUNK):
        sv = sums[pl.ds(g * 16, 16)]
        row0[pl.ds(g * 16, 16)] = sv * a0 + c0
        row1[pl.ds(g * 16, 16)] = sv * a1 + c1

    pltpu.sync_copy(row0, out0_h.at[pl.ds(wid * _DOCS_PER_W, _DOCS_PER_W)])
    pltpu.sync_copy(row1, out1_h.at[pl.ds(wid * _DOCS_PER_W, _DOCS_PER_W)])


_mesh = plsc.VectorSubcoreMesh(core_axis_name="c", subcore_axis_name="s")

_sc_call = functools.partial(
    pl.kernel,
    mesh=_mesh,
    compiler_params=pltpu.CompilerParams(needs_layout_passes=False),
    out_type=[
        jax.ShapeDtypeStruct((_N,), jnp.float32),
        jax.ShapeDtypeStruct((_N,), jnp.float32),
    ],
    scratch_types=[
        pltpu.VMEM((_TAB,), jnp.float32),
        pltpu.VMEM((64,), jnp.float32),
        pltpu.VMEM((_CHUNK_WORDS,), jnp.float32),
        pltpu.VMEM((_CHUNK_WORDS,), jnp.float32),
        pltpu.VMEM((_DOCS_PER_W,), jnp.float32),
        pltpu.VMEM((_DOCS_PER_W,), jnp.float32),
        pltpu.VMEM((_DOCS_PER_W,), jnp.float32),
        pltpu.SemaphoreType.DMA,
        pltpu.SemaphoreType.DMA,
    ],
)(_sc_body)


def kernel(images, W_mlp, b_mlp, W_final, b_final):
    flat = images.reshape(-1)
    wtab = jnp.concatenate(
        [W_mlp[:, 0], jnp.zeros((_TAB - _N_DEPTH * _N_BINS,), jnp.float32)])
    wsum = W_final[0] + W_final[1] + W_final[2]
    a = wsum / jnp.float32(_N_GEN)
    cc = b_mlp[0] * wsum + b_final
    params = jnp.concatenate([
        jnp.full((16,), a[0], jnp.float32),
        jnp.full((16,), a[1], jnp.float32),
        jnp.full((16,), cc[0], jnp.float32),
        jnp.full((16,), cc[1], jnp.float32),
    ])
    out0, out1 = _sc_call(flat, wtab, params)
    return jnp.stack([out0, out1], axis=-1)


# speedup vs baseline: 396.7614x; 13.1874x over previous
import functools

import jax
import jax.numpy as jnp
from jax import lax
from jax.experimental import pallas as pl
from jax.experimental.pallas import tpu as pltpu
from jax.experimental.pallas import tpu_sc as plsc

_N = 4096
_N_DEPTH = 3
_N_ORI = 100
_N_GEN = 10
_N_BINS = 50
_NPLANE = _N_DEPTH * _N_GEN
_NW = 32
_DOCS_PER_W = _N // _NW
_NQ = _DOCS_PER_W // 16
_TAB = 160


def _sc_body(planes_h, wtab_h, params_h, out0_h, out1_h,
             wtab, params, buf0, buf1, sums, row0, row1, sem0, sem1):
    c = lax.axis_index("c")
    s = lax.axis_index("s")
    wid = s * 2 + c
    col = wid * _DOCS_PER_W

    pltpu.sync_copy(wtab_h, wtab)
    pltpu.sync_copy(params_h, params)

    bufs = (buf0, buf1)
    sems = (sem0, sem1)
    handles = [None, None]
    handles[0] = pltpu.async_copy(
        planes_h.at[0, 0, slice(None), pl.ds(col, _DOCS_PER_W)],
        bufs[0], sems[0])

    accs = tuple(jnp.zeros((16,), jnp.float32) for _ in range(_NQ))

    for p in range(_NPLANE):
        b = p & 1
        handles[b].wait()
        if p + 1 < _NPLANE:
            d1, g1 = divmod(p + 1, _N_GEN)
            handles[1 - b] = pltpu.async_copy(
                planes_h.at[d1, g1, slice(None), pl.ds(col, _DOCS_PER_W)],
                bufs[1 - b], sems[1 - b])

        buf = bufs[b]
        off = (p // _N_GEN) * _N_BINS

        @plsc.parallel_loop(0, _N_ORI, unroll=2, carry=accs)
        def body(r, acc_t, _buf=buf, _off=off):
            new = []
            for q in range(_NQ):
                x = _buf[r, pl.ds(16 * q, 16)]
                t = jnp.minimum(x * jnp.float32(_N_BINS),
                                jnp.float32(_N_BINS - 1))
                ti = t.astype(jnp.int32)
                if _off:
                    ti = ti + _off
                w = plsc.load_gather(wtab, [ti])
                new.append(acc_t[q] + w)
            return tuple(new)

        accs = body

    a0 = params[pl.ds(0, 16)]
    a1 = params[pl.ds(16, 16)]
    c0 = params[pl.ds(32, 16)]
    c1 = params[pl.ds(48, 16)]
    for q in range(_NQ):
        row0[pl.ds(q * 16, 16)] = accs[q] * a0 + c0
        row1[pl.ds(q * 16, 16)] = accs[q] * a1 + c1

    pltpu.sync_copy(row0, out0_h.at[pl.ds(col, _DOCS_PER_W)])
    pltpu.sync_copy(row1, out1_h.at[pl.ds(col, _DOCS_PER_W)])


_mesh = plsc.VectorSubcoreMesh(core_axis_name="c", subcore_axis_name="s")

_sc_call = functools.partial(
    pl.kernel,
    mesh=_mesh,
    compiler_params=pltpu.CompilerParams(needs_layout_passes=False),
    out_type=[
        jax.ShapeDtypeStruct((_N,), jnp.float32),
        jax.ShapeDtypeStruct((_N,), jnp.float32),
    ],
    scratch_types=[
        pltpu.VMEM((_TAB,), jnp.float32),
        pltpu.VMEM((64,), jnp.float32),
        pltpu.VMEM((_N_ORI, _DOCS_PER_W), jnp.float32),
        pltpu.VMEM((_N_ORI, _DOCS_PER_W), jnp.float32),
        pltpu.VMEM((_DOCS_PER_W,), jnp.float32),
        pltpu.VMEM((_DOCS_PER_W,), jnp.float32),
        pltpu.VMEM((_DOCS_PER_W,), jnp.float32),
        pltpu.SemaphoreType.DMA,
        pltpu.SemaphoreType.DMA,
    ],
)(_sc_body)


def kernel(images, W_mlp, b_mlp, W_final, b_final):
    planes = jnp.transpose(images, (1, 3, 2, 0))
    wtab = jnp.concatenate(
        [W_mlp[:, 0], jnp.zeros((_TAB - _N_DEPTH * _N_BINS,), jnp.float32)])
    wsum = W_final[0] + W_final[1] + W_final[2]
    a = wsum / jnp.float32(_N_GEN)
    cc = b_mlp[0] * wsum + b_final
    params = jnp.concatenate([
        jnp.full((16,), a[0], jnp.float32),
        jnp.full((16,), a[1], jnp.float32),
        jnp.full((16,), cc[0], jnp.float32),
        jnp.full((16,), cc[1], jnp.float32),
    ])
    out0, out1 = _sc_call(planes, wtab, params)
    return jnp.stack([out0, out1], axis=-1)
